# baseline (speedup 1.0000x reference)
import functools

import jax
import jax.numpy as jnp
from jax import lax
from jax.experimental import pallas as pl
from jax.experimental.pallas import tpu as pltpu
from jax.experimental.pallas import tpu_sc as plsc

VOCAB = 1000000
DIM = 64
B_POS = 16384
B_NEG = 81920
B_TOT = B_POS + B_NEG

NC = 2
NS = 16
NW = NC * NS
PER_W = B_TOT // NW
CH = 192
NCH = PER_W // CH
HALF = CH // 2
GROUPS = CH // 16


def _sc_dot_kernel(physv_hbm, physu_hbm, colv_hbm, colu_hbm, vtab, utab,
                   out_hbm,
                   physv_all, physu_all, colv_all, colu_all,
                   va, ua, vb, ub, zbuf,
                   semva, semua, semvb, semub):
    wid = lax.axis_index("s") * NC + lax.axis_index("c")
    pltpu.sync_copy(physv_hbm.at[wid], physv_all)
    pltpu.sync_copy(physu_hbm.at[wid], physu_all)
    pltpu.sync_copy(colv_hbm.at[wid], colv_all)
    pltpu.sync_copy(colu_hbm.at[wid], colu_all)

    bufs = ((va, ua, semva, semua), (vb, ub, semvb, semub))

    def issue(t, bufset):
        vB, uB, sv, su = bufset
        for h in range(2):
            pltpu.async_copy(
                vtab.at[physv_all.at[t, h]], vB.at[pl.ds(h * HALF, HALF)], sv)
            pltpu.async_copy(
                utab.at[physu_all.at[t, h]], uB.at[pl.ds(h * HALF, HALF)], su)

    def drain(bufset):
        vB, uB, sv, su = bufset
        for h in range(2):
            pltpu.make_async_copy(
                vtab.at[pl.ds(0, HALF)], vB.at[pl.ds(h * HALF, HALF)], sv
            ).wait()
            pltpu.make_async_copy(
                utab.at[pl.ds(0, HALF)], uB.at[pl.ds(h * HALF, HALF)], su
            ).wait()

    def compute(t, bufset):
        vB, uB = bufset[0], bufset[1]

        def gbody(g, carry):
            rows = g * 16 + lax.broadcasted_iota(jnp.int32, (16,), 0)
            base = pl.multiple_of(t * CH + g * 16, 16)
            cv = colv_all[pl.ds(base, 16)]
            cu = colu_all[pl.ds(base, 16)]
            acc = jnp.zeros((16,), jnp.float32)
            for c in range(DIM):
                a = plsc.load_gather(vB, [rows, cv + c])
                b = plsc.load_gather(uB, [rows, cu + c])
                acc = acc + a * b
            zbuf[pl.ds(pl.multiple_of(g * 16, 16), 16)] = acc
            return carry

        lax.fori_loop(0, GROUPS, gbody, jnp.int32(0))
        pltpu.sync_copy(zbuf, out_hbm.at[wid, t])

    issue(0, bufs[0])
    issue(1, bufs[1])

    def chunk_pair(tp, carry):
        t0 = tp * 2
        drain(bufs[0])
        compute(t0, bufs[0])
        issue(jnp.minimum(t0 + 2, NCH - 1), bufs[0])
        drain(bufs[1])
        compute(t0 + 1, bufs[1])
        issue(jnp.minimum(t0 + 3, NCH - 1), bufs[1])
        return carry

    lax.fori_loop(0, NCH // 2, chunk_pair, jnp.int32(0))
    drain(bufs[0])
    drain(bufs[1])


def _sc_dot(physv, physu, colv, colu, vt2, ut2):
    mesh = plsc.VectorSubcoreMesh(core_axis_name="c", subcore_axis_name="s")
    k = functools.partial(
        pl.kernel,
        mesh=mesh,
        compiler_params=pltpu.CompilerParams(
            needs_layout_passes=False, use_tc_tiling_on_sc=True),
        out_type=jax.ShapeDtypeStruct((NW, NCH, CH), jnp.float32),
        scratch_types=[
            pltpu.VMEM((NCH, 2, HALF), jnp.int32),
            pltpu.VMEM((NCH, 2, HALF), jnp.int32),
            pltpu.VMEM((PER_W,), jnp.int32),
            pltpu.VMEM((PER_W,), jnp.int32),
            pltpu.VMEM((CH, 128), jnp.float32),
            pltpu.VMEM((CH, 128), jnp.float32),
            pltpu.VMEM((CH, 128), jnp.float32),
            pltpu.VMEM((CH, 128), jnp.float32),
            pltpu.VMEM((CH,), jnp.float32),
            pltpu.SemaphoreType.DMA,
            pltpu.SemaphoreType.DMA,
            pltpu.SemaphoreType.DMA,
            pltpu.SemaphoreType.DMA,
        ],
    )(_sc_dot_kernel)
    return k(physv, physu, colv, colu, vt2, ut2)


def _loss_body(z_ref, o_ref):
    z = z_ref[...]
    rows = lax.broadcasted_iota(jnp.int32, z.shape, 0)
    sign = jnp.where(rows < (B_POS // 128), 1.0, -1.0)
    x = sign * z
    a = jnp.minimum(x, 0.0) - jnp.log1p(jnp.exp(-jnp.abs(x)))
    o_ref[0, 0] = -jnp.sum(a)


def kernel(pos_v, pos_u, neg_v, neg_u, v_table, u_table):
    idx_v = jnp.concatenate([pos_v, neg_v]).astype(jnp.int32)
    idx_u = jnp.concatenate([pos_u, neg_u]).astype(jnp.int32)
    vt2 = jnp.reshape(v_table, (VOCAB // 2, 2 * DIM))
    ut2 = jnp.reshape(u_table, (VOCAB // 2, 2 * DIM))
    physv = (idx_v >> 1).reshape(NW, NCH, 2, HALF)
    physu = (idx_u >> 1).reshape(NW, NCH, 2, HALF)
    colv = ((idx_v & 1) * DIM).reshape(NW, PER_W)
    colu = ((idx_u & 1) * DIM).reshape(NW, PER_W)
    z = _sc_dot(physv, physu, colv, colu, vt2, ut2)
    z2 = z.reshape(B_TOT // 128, 128)
    loss = pl.pallas_call(
        _loss_body,
        out_shape=jax.ShapeDtypeStruct((1, 1), jnp.float32),
        out_specs=pl.BlockSpec(memory_space=pltpu.SMEM),
    )(z2)
    return loss[0, 0]


# speedup vs baseline: 1.4036x; 1.4036x over previous
import functools

import jax
import jax.numpy as jnp
from jax import lax
from jax.experimental import pallas as pl
from jax.experimental.pallas import tpu as pltpu
from jax.experimental.pallas import tpu_sc as plsc

VOCAB = 1000000
DIM = 64
B_POS = 16384
B_NEG = 81920
B_TOT = B_POS + B_NEG

NC = 2
NS = 16
NW = NC * NS
PER_W = B_TOT // NW
CH = 192
NCH = PER_W // CH
HALF = CH // 2
GROUPS = CH // 16


def _sc_dot_kernel(idxv_hbm, idxu_hbm, tab, out_hbm,
                   idxv_all, idxu_all,
                   va, ua, vb, ub, zbuf,
                   semva, semua, semvb, semub):
    wid = lax.axis_index("s") * NC + lax.axis_index("c")
    pltpu.sync_copy(idxv_hbm.at[wid], idxv_all)
    pltpu.sync_copy(idxu_hbm.at[wid], idxu_all)

    bufs = ((va, ua, semva, semua), (vb, ub, semvb, semub))

    def issue(t, bufset):
        vB, uB, sv, su = bufset
        for h in range(2):
            pltpu.async_copy(
                tab.at[idxv_all.at[t, h]], vB.at[pl.ds(h * HALF, HALF)], sv)
            pltpu.async_copy(
                tab.at[idxu_all.at[t, h]], uB.at[pl.ds(h * HALF, HALF)], su)

    def drain(bufset):
        vB, uB, sv, su = bufset
        for h in range(2):
            pltpu.make_async_copy(
                tab.at[pl.ds(0, HALF)], vB.at[pl.ds(h * HALF, HALF)], sv
            ).wait()
            pltpu.make_async_copy(
                tab.at[pl.ds(0, HALF)], uB.at[pl.ds(h * HALF, HALF)], su
            ).wait()

    def compute(t, bufset):
        vB, uB = bufset[0], bufset[1]

        def gbody(g, carry):
            lane = lax.broadcasted_iota(jnp.int32, (16,), 0)
            acc = jnp.zeros((16,), jnp.float32)
            for j in range(16):
                r = g * 16 + j
                prod = jnp.zeros((16,), jnp.float32)
                for k in range(DIM // 16):
                    a = vB[r, pl.ds(k * 16, 16)]
                    b = uB[r, pl.ds(DIM + k * 16, 16)]
                    prod = prod + a * b
                s = jnp.sum(prod)
                acc = jnp.where(lane == j, s, acc)
            zbuf[pl.ds(pl.multiple_of(g * 16, 16), 16)] = acc
            return carry

        lax.fori_loop(0, GROUPS, gbody, jnp.int32(0))
        pltpu.sync_copy(zbuf, out_hbm.at[wid, t])

    issue(0, bufs[0])
    issue(1, bufs[1])

    def chunk_pair(tp, carry):
        t0 = tp * 2
        drain(bufs[0])
        compute(t0, bufs[0])
        issue(jnp.minimum(t0 + 2, NCH - 1), bufs[0])
        drain(bufs[1])
        compute(t0 + 1, bufs[1])
        issue(jnp.minimum(t0 + 3, NCH - 1), bufs[1])
        return carry

    lax.fori_loop(0, NCH // 2, chunk_pair, jnp.int32(0))
    drain(bufs[0])
    drain(bufs[1])


def _sc_dot(idxv, idxu, tab2):
    mesh = plsc.VectorSubcoreMesh(core_axis_name="c", subcore_axis_name="s")
    k = functools.partial(
        pl.kernel,
        mesh=mesh,
        compiler_params=pltpu.CompilerParams(
            needs_layout_passes=False, use_tc_tiling_on_sc=True),
        out_type=jax.ShapeDtypeStruct((NW, NCH, CH), jnp.float32),
        scratch_types=[
            pltpu.VMEM((NCH, 2, HALF), jnp.int32),
            pltpu.VMEM((NCH, 2, HALF), jnp.int32),
            pltpu.VMEM((CH, 128), jnp.float32),
            pltpu.VMEM((CH, 128), jnp.float32),
            pltpu.VMEM((CH, 128), jnp.float32),
            pltpu.VMEM((CH, 128), jnp.float32),
            pltpu.VMEM((CH,), jnp.float32),
            pltpu.SemaphoreType.DMA,
            pltpu.SemaphoreType.DMA,
            pltpu.SemaphoreType.DMA,
            pltpu.SemaphoreType.DMA,
        ],
    )(_sc_dot_kernel)
    return k(idxv, idxu, tab2)


def _loss_body(z_ref, o_ref):
    z = z_ref[...]
    rows = lax.broadcasted_iota(jnp.int32, z.shape, 0)
    sign = jnp.where(rows < (B_POS // 128), 1.0, -1.0)
    x = sign * z
    a = jnp.minimum(x, 0.0) - jnp.log1p(jnp.exp(-jnp.abs(x)))
    o_ref[0, 0] = -jnp.sum(a)


def kernel(pos_v, pos_u, neg_v, neg_u, v_table, u_table):
    idx_v = jnp.concatenate([pos_v, neg_v]).astype(jnp.int32)
    idx_u = jnp.concatenate([pos_u, neg_u]).astype(jnp.int32)
    tab2 = jnp.concatenate([v_table, u_table], axis=1)
    idxv = idx_v.reshape(NW, NCH, 2, HALF)
    idxu = idx_u.reshape(NW, NCH, 2, HALF)
    z = _sc_dot(idxv, idxu, tab2)
    z2 = z.reshape(B_TOT // 128, 128)
    loss = pl.pallas_call(
        _loss_body,
        out_shape=jax.ShapeDtypeStruct((1, 1), jnp.float32),
        out_specs=pl.BlockSpec(memory_space=pltpu.SMEM),
    )(z2)
    return loss[0, 0]


# speedup vs baseline: 1.8749x; 1.3358x over previous
import functools

import jax
import jax.numpy as jnp
from jax import lax
from jax.experimental import pallas as pl
from jax.experimental.pallas import tpu as pltpu
from jax.experimental.pallas import tpu_sc as plsc

VOCAB = 1000000
DIM = 64
B_POS = 16384
B_NEG = 81920
B_TOT = B_POS + B_NEG

NC = 2
NS = 16
NW = NC * NS
PER_W = B_TOT // NW
CH = 192
NCH = PER_W // CH
HALF = CH // 2
GROUPS = CH // 16


def _sc_dot_kernel(idxv_hbm, idxu_hbm, tab, out_hbm,
                   idxv_all, idxu_all,
                   va, ua, vb, ub, zbuf,
                   semva, semua, semvb, semub):
    wid = lax.axis_index("s") * NC + lax.axis_index("c")
    pltpu.sync_copy(idxv_hbm.at[wid], idxv_all)
    pltpu.sync_copy(idxu_hbm.at[wid], idxu_all)

    bufs = ((va, ua, semva, semua), (vb, ub, semvb, semub))

    def issue(t, bufset):
        vB, uB, sv, su = bufset
        for h in range(2):
            pltpu.async_copy(
                tab.at[idxv_all.at[t, h]], vB.at[pl.ds(h * HALF, HALF)], sv)
            pltpu.async_copy(
                tab.at[idxu_all.at[t, h]], uB.at[pl.ds(h * HALF, HALF)], su)

    def drain(bufset):
        vB, uB, sv, su = bufset
        for h in range(2):
            pltpu.make_async_copy(
                tab.at[pl.ds(0, HALF)], vB.at[pl.ds(h * HALF, HALF)], sv
            ).wait()
            pltpu.make_async_copy(
                tab.at[pl.ds(0, HALF)], uB.at[pl.ds(h * HALF, HALF)], su
            ).wait()

    def compute(t, bufset):
        vB, uB = bufset[0], bufset[1]

        def gbody(g, carry):
            lane = lax.broadcasted_iota(jnp.int32, (16,), 0)
            acc = jnp.zeros((16,), jnp.float32)
            for j in range(16):
                r = g * 16 + j
                prod = jnp.zeros((16,), jnp.float32)
                for k in range(DIM // 16):
                    a = vB[r, pl.ds(k * 16, 16)]
                    b = uB[r, pl.ds(DIM + k * 16, 16)]
                    prod = prod + a * b
                s = jnp.sum(prod)
                acc = jnp.where(lane == j, s, acc)
            zbuf[pl.ds(pl.multiple_of(g * 16, 16), 16)] = acc
            return carry

        lax.fori_loop(0, GROUPS, gbody, jnp.int32(0))
        pltpu.sync_copy(zbuf, out_hbm.at[wid, t])

    issue(0, bufs[0])
    issue(1, bufs[1])

    def chunk_pair(tp, carry):
        t0 = tp * 2
        drain(bufs[0])
        compute(t0, bufs[0])
        issue(jnp.minimum(t0 + 2, NCH - 1), bufs[0])
        drain(bufs[1])
        compute(t0 + 1, bufs[1])
        issue(jnp.minimum(t0 + 3, NCH - 1), bufs[1])
        return carry

    lax.fori_loop(0, NCH // 2, chunk_pair, jnp.int32(0))
    drain(bufs[0])
    drain(bufs[1])


def _sc_dot(idxv, idxu, tab2):
    mesh = plsc.VectorSubcoreMesh(core_axis_name="c", subcore_axis_name="s")
    k = functools.partial(
        pl.kernel,
        mesh=mesh,
        compiler_params=pltpu.CompilerParams(
            needs_layout_passes=False, use_tc_tiling_on_sc=True),
        out_type=jax.ShapeDtypeStruct((NW, NCH, CH), jnp.float32),
        scratch_types=[
            pltpu.VMEM((NCH, 2, HALF), jnp.int32),
            pltpu.VMEM((NCH, 2, HALF), jnp.int32),
            pltpu.VMEM((CH, 128), jnp.float32),
            pltpu.VMEM((CH, 128), jnp.float32),
            pltpu.VMEM((CH, 128), jnp.float32),
            pltpu.VMEM((CH, 128), jnp.float32),
            pltpu.VMEM((CH,), jnp.float32),
            pltpu.SemaphoreType.DMA,
            pltpu.SemaphoreType.DMA,
            pltpu.SemaphoreType.DMA,
            pltpu.SemaphoreType.DMA,
        ],
    )(_sc_dot_kernel)
    return k(idxv, idxu, tab2)


TRW = 2048


def _merge_body(v_ref, u_ref, o_ref):
    tv = v_ref[...].T
    tu = u_ref[...].T
    o_ref[...] = jnp.concatenate([tv, tu], axis=1)


def _merge_transpose(v_table, u_table):
    vtT = v_table.T
    utT = u_table.T
    nblk = (VOCAB + TRW - 1) // TRW
    return pl.pallas_call(
        _merge_body,
        grid=(nblk,),
        in_specs=[
            pl.BlockSpec((DIM, TRW), lambda i: (0, i)),
            pl.BlockSpec((DIM, TRW), lambda i: (0, i)),
        ],
        out_specs=pl.BlockSpec((TRW, 2 * DIM), lambda i: (i, 0)),
        out_shape=jax.ShapeDtypeStruct((VOCAB, 2 * DIM), jnp.float32),
    )(vtT, utT)


def _loss_body(z_ref, o_ref):
    z = z_ref[...]
    rows = lax.broadcasted_iota(jnp.int32, z.shape, 0)
    sign = jnp.where(rows < (B_POS // 128), 1.0, -1.0)
    x = sign * z
    a = jnp.minimum(x, 0.0) - jnp.log1p(jnp.exp(-jnp.abs(x)))
    o_ref[0, 0] = -jnp.sum(a)


def kernel(pos_v, pos_u, neg_v, neg_u, v_table, u_table):
    idx_v = jnp.concatenate([pos_v, neg_v]).astype(jnp.int32)
    idx_u = jnp.concatenate([pos_u, neg_u]).astype(jnp.int32)
    tab2 = _merge_transpose(v_table, u_table)
    idxv = idx_v.reshape(NW, NCH, 2, HALF)
    idxu = idx_u.reshape(NW, NCH, 2, HALF)
    z = _sc_dot(idxv, idxu, tab2)
    z2 = z.reshape(B_TOT // 128, 128)
    loss = pl.pallas_call(
        _loss_body,
        out_shape=jax.ShapeDtypeStruct((1, 1), jnp.float32),
        out_specs=pl.BlockSpec(memory_space=pltpu.SMEM),
    )(z2)
    return loss[0, 0]


# speedup vs baseline: 2.3114x; 1.2329x over previous
import functools

import jax
import jax.numpy as jnp
from jax import lax
from jax.experimental import pallas as pl
from jax.experimental.pallas import tpu as pltpu
from jax.experimental.pallas import tpu_sc as plsc

VOCAB = 1000000
DIM = 64
B_POS = 16384
B_NEG = 81920
B_TOT = B_POS + B_NEG

NC = 2
NS = 16
NW = NC * NS
PER_W = B_TOT // NW
CH = 192
NCH = PER_W // CH
HALF = CH // 2
GROUPS = CH // 16


def _sc_dot_kernel(idxv_hbm, idxu_hbm, tab, out_hbm,
                   idxv_all, idxu_all,
                   va, ua, vb, ub, zbuf,
                   semva, semua, semvb, semub):
    wid = lax.axis_index("s") * NC + lax.axis_index("c")
    pltpu.sync_copy(idxv_hbm.at[wid], idxv_all)
    pltpu.sync_copy(idxu_hbm.at[wid], idxu_all)

    bufs = ((va, ua, semva, semua), (vb, ub, semvb, semub))

    def issue(t, bufset):
        vB, uB, sv, su = bufset
        for h in range(2):
            pltpu.async_copy(
                tab.at[idxv_all.at[t, h]], vB.at[pl.ds(h * HALF, HALF)], sv)
            pltpu.async_copy(
                tab.at[idxu_all.at[t, h]], uB.at[pl.ds(h * HALF, HALF)], su)

    def drain(bufset):
        vB, uB, sv, su = bufset
        for h in range(2):
            pltpu.make_async_copy(
                tab.at[pl.ds(0, HALF)], vB.at[pl.ds(h * HALF, HALF)], sv
            ).wait()
            pltpu.make_async_copy(
                tab.at[pl.ds(0, HALF)], uB.at[pl.ds(h * HALF, HALF)], su
            ).wait()

    def compute(t, bufset):
        vB, uB = bufset[0], bufset[1]

        def gbody(g, carry):
            lane = lax.broadcasted_iota(jnp.int32, (16,), 0)
            acc = jnp.zeros((16,), jnp.float32)
            for j in range(16):
                r = g * 16 + j
                prod = jnp.zeros((16,), jnp.float32)
                for k in range(DIM // 16):
                    a = vB[r, pl.ds(k * 16, 16)]
                    b = uB[r, pl.ds(DIM + k * 16, 16)]
                    prod = prod + a * b
                s = jnp.sum(prod)
                acc = jnp.where(lane == j, s, acc)
            zbuf[pl.ds(pl.multiple_of(g * 16, 16), 16)] = acc
            return carry

        lax.fori_loop(0, GROUPS, gbody, jnp.int32(0))
        pltpu.sync_copy(zbuf, out_hbm.at[wid, t])

    issue(0, bufs[0])
    issue(1, bufs[1])

    def chunk_pair(tp, carry):
        t0 = tp * 2
        drain(bufs[0])
        compute(t0, bufs[0])
        issue(jnp.minimum(t0 + 2, NCH - 1), bufs[0])
        drain(bufs[1])
        compute(t0 + 1, bufs[1])
        issue(jnp.minimum(t0 + 3, NCH - 1), bufs[1])
        return carry

    lax.fori_loop(0, NCH // 2, chunk_pair, jnp.int32(0))
    drain(bufs[0])
    drain(bufs[1])


def _sc_dot(idxv, idxu, tab2):
    mesh = plsc.VectorSubcoreMesh(core_axis_name="c", subcore_axis_name="s")
    k = functools.partial(
        pl.kernel,
        mesh=mesh,
        compiler_params=pltpu.CompilerParams(
            needs_layout_passes=False, use_tc_tiling_on_sc=True),
        out_type=jax.ShapeDtypeStruct((NW, NCH, CH), jnp.float32),
        scratch_types=[
            pltpu.VMEM((NCH, 2, HALF), jnp.int32),
            pltpu.VMEM((NCH, 2, HALF), jnp.int32),
            pltpu.VMEM((CH, 128), jnp.float32),
            pltpu.VMEM((CH, 128), jnp.float32),
            pltpu.VMEM((CH, 128), jnp.float32),
            pltpu.VMEM((CH, 128), jnp.float32),
            pltpu.VMEM((CH,), jnp.float32),
            pltpu.SemaphoreType.DMA,
            pltpu.SemaphoreType.DMA,
            pltpu.SemaphoreType.DMA,
            pltpu.SemaphoreType.DMA,
        ],
    )(_sc_dot_kernel)
    return k(idxv, idxu, tab2)


TRW = 4096


def _merge_body(v_ref, u_ref, o_ref):
    o_ref[:, 0:DIM] = v_ref[...].T
    o_ref[:, DIM:2 * DIM] = u_ref[...].T


def _merge_transpose(v_table, u_table):
    vtT = v_table.T
    utT = u_table.T
    nblk = (VOCAB + TRW - 1) // TRW
    return pl.pallas_call(
        _merge_body,
        grid=(nblk,),
        in_specs=[
            pl.BlockSpec((DIM, TRW), lambda i: (0, i)),
            pl.BlockSpec((DIM, TRW), lambda i: (0, i)),
        ],
        out_specs=pl.BlockSpec((TRW, 2 * DIM), lambda i: (i, 0)),
        out_shape=jax.ShapeDtypeStruct((VOCAB, 2 * DIM), jnp.float32),
        compiler_params=pltpu.CompilerParams(
            dimension_semantics=("parallel",)),
    )(vtT, utT)


def _loss_body(z_ref, o_ref):
    z = z_ref[...]
    rows = lax.broadcasted_iota(jnp.int32, z.shape, 0)
    sign = jnp.where(rows < (B_POS // 128), 1.0, -1.0)
    x = sign * z
    a = jnp.minimum(x, 0.0) - jnp.log1p(jnp.exp(-jnp.abs(x)))
    o_ref[0, 0] = -jnp.sum(a)


def kernel(pos_v, pos_u, neg_v, neg_u, v_table, u_table):
    idx_v = jnp.concatenate([pos_v, neg_v]).astype(jnp.int32)
    idx_u = jnp.concatenate([pos_u, neg_u]).astype(jnp.int32)
    tab2 = _merge_transpose(v_table, u_table)
    idxv = idx_v.reshape(NW, NCH, 2, HALF)
    idxu = idx_u.reshape(NW, NCH, 2, HALF)
    z = _sc_dot(idxv, idxu, tab2)
    z2 = z.reshape(B_TOT // 128, 128)
    loss = pl.pallas_call(
        _loss_body,
        out_shape=jax.ShapeDtypeStruct((1, 1), jnp.float32),
        out_specs=pl.BlockSpec(memory_space=pltpu.SMEM),
    )(z2)
    return loss[0, 0]
